# baseline (speedup 1.0000x reference)
import jax
import jax.numpy as jnp
from jax import lax
from jax.experimental import pallas as pl
from jax.experimental.pallas import tpu as pltpu
from jax.experimental.pallas import tpu_sc as plsc

VOCAB = 100000
EMB = 128
OUT = 2
HID = (EMB + OUT) // 2
SEQ = 200
BATCH = 4096

NC = 2
NS = 16
L = 16
NW = NC * NS
ROWS_PER_W = BATCH // NW
WORDS_PER_W = ROWS_PER_W * SEQ
FULL_CHUNKS = SEQ // L
TAIL = SEQ - FULL_CHUNKS * L
CHUNK_ROWS = 32
NCHUNKS = ROWS_PER_W // CHUNK_ROWS
CHUNK_WORDS = CHUNK_ROWS * SEQ

PROJ_CHUNK = 128
PROJ_CHUNKS_PER_TILE = 26
EMB_WORDS = PROJ_CHUNK * EMB
KV = EMB // L


def _wc_body(w1_ref, w2_ref, wc_ref):
    wc_ref[...] = lax.dot_general(w2_ref[...], w1_ref[...],
                                  (((1,), (0,)), ((), ())),
                                  preferred_element_type=jnp.float32)


def _round_bf16(u):
    return lax.shift_right_logical(
        u + jnp.uint32(0x7FFF)
        + (lax.shift_right_logical(u, jnp.uint32(16)) & jnp.uint32(1)),
        jnp.uint32(16))


def _proj_body(emb_hbm, wc_hbm, p_hbm, ebuf0, ebuf1, pbuf, wcv, sem0, sem1):
    c = lax.axis_index("c")
    s = lax.axis_index("s")
    wid = s * NC + c
    lanes = lax.iota(jnp.int32, L)
    last_mask = lanes == (L - 1)
    pltpu.sync_copy(wc_hbm, wcv)
    w0 = [wcv[pl.ds(k * L, L)] for k in range(KV)]
    w1 = [wcv[pl.ds(EMB + k * L, L)] for k in range(KV)]

    def chunk_start(ci):
        return jnp.minimum((wid + ci * NW) * PROJ_CHUNK, VOCAB - PROJ_CHUNK)

    def copy_op(ci, ebuf, sem):
        st = chunk_start(ci)
        return pltpu.make_async_copy(
            emb_hbm.at[pl.ds(st * EMB, EMB_WORDS)], ebuf, sem)

    def do_chunk(ci, ebuf, sem):
        st = chunk_start(ci)
        copy_op(ci, ebuf, sem).wait()

        @plsc.parallel_loop(0, PROJ_CHUNK, unroll=4)
        def row(r):
            roff = r * EMB
            acc0 = jnp.zeros((L,), jnp.float32)
            acc1 = jnp.zeros((L,), jnp.float32)
            for k in range(KV):
                e = ebuf[pl.ds(roff + k * L, L)]
                acc0 = acc0 + e * w0[k]
                acc1 = acc1 + e * w1[k]
            u0 = plsc.bitcast(plsc.cumsum(acc0), jnp.uint32)
            u1 = plsc.bitcast(plsc.cumsum(acc1), jnp.uint32)
            packed = plsc.bitcast(
                _round_bf16(u0)
                | lax.shift_left(_round_bf16(u1), jnp.uint32(16)), jnp.int32)
            plsc.store_scatter(pbuf, [jnp.full((L,), r, jnp.int32)],
                               packed, mask=last_mask)

        pltpu.sync_copy(pbuf, p_hbm.at[pl.ds(st, PROJ_CHUNK)])

    copy_op(0, ebuf0, sem0).start()

    @pl.loop(0, PROJ_CHUNKS_PER_TILE, step=2)
    def pair(ci):
        copy_op(ci + 1, ebuf1, sem1).start()
        do_chunk(ci, ebuf0, sem0)
        copy_op(ci + 2, ebuf0, sem0).start()
        do_chunk(ci + 1, ebuf1, sem1)

    copy_op(PROJ_CHUNKS_PER_TILE, ebuf0, sem0).wait()


def _unpack_pair(word):
    v0 = plsc.bitcast(lax.shift_left(word, 16), jnp.float32)
    v1 = plsc.bitcast(jnp.bitwise_and(word, jnp.int32(-65536)), jnp.float32)
    return v0, v1


def _pool_body(text_hbm, p_hbm, bias_hbm, out_hbm, text_v, pvals, rowacc0,
               rowacc1, out_v, bias_v):
    c = lax.axis_index("c")
    s = lax.axis_index("s")
    wid = s * NC + c
    wbase = wid * WORDS_PER_W
    tail_mask = lax.iota(jnp.int32, L) < TAIL
    lanes = lax.iota(jnp.int32, L)
    pltpu.sync_copy(bias_hbm, bias_v)
    pltpu.sync_copy(p_hbm, pvals)

    for cb in range(NCHUNKS):
        pltpu.sync_copy(
            text_hbm.at[pl.ds(wbase + cb * CHUNK_WORDS, CHUNK_WORDS)],
            text_v.at[pl.ds(0, CHUNK_WORDS)])
        text_v[pl.ds(CHUNK_WORDS, L)] = jnp.zeros((L,), jnp.int32)

        @plsc.parallel_loop(0, CHUNK_ROWS, unroll=4)
        def row_body(r):
            rbase = r * SEQ
            acc0 = jnp.zeros((L,), jnp.float32)
            acc1 = jnp.zeros((L,), jnp.float32)
            for ck in range(FULL_CHUNKS):
                idx = text_v[pl.ds(rbase + ck * L, L)]
                v0, v1 = _unpack_pair(plsc.load_gather(pvals, [idx]))
                acc0 = acc0 + v0
                acc1 = acc1 + v1
            tidx = text_v[pl.ds(rbase + FULL_CHUNKS * L, L)]
            tv0, tv1 = _unpack_pair(plsc.load_gather(pvals, [tidx]))
            acc0 = acc0 + jnp.where(tail_mask, tv0, 0.0)
            acc1 = acc1 + jnp.where(tail_mask, tv1, 0.0)
            rowacc0[pl.ds(r * L, L)] = acc0
            rowacc1[pl.ds(r * L, L)] = acc1

        @plsc.parallel_loop(0, CHUNK_ROWS // L, unroll=2)
        def grp_body(g, cb=cb):
            base = g * L * L
            tot0 = jnp.zeros((L,), jnp.float32)
            tot1 = jnp.zeros((L,), jnp.float32)
            for l in range(L):
                tot0 = tot0 + plsc.load_gather(rowacc0, [base + lanes * L + l])
                tot1 = tot1 + plsc.load_gather(rowacc1, [base + lanes * L + l])
            rows = cb * CHUNK_ROWS + g * L + lanes
            plsc.store_scatter(
                out_v, [rows, jnp.zeros((L,), jnp.int32)],
                tot0 * (1.0 / SEQ) + bias_v[pl.ds(0, L)])
            plsc.store_scatter(
                out_v, [rows, jnp.full((L,), 1, jnp.int32)],
                tot1 * (1.0 / SEQ) + bias_v[pl.ds(L, L)])

    rowbase = wid * ROWS_PER_W
    pltpu.sync_copy(out_v, out_hbm.at[pl.ds(rowbase, ROWS_PER_W), :])


_SC_MESH = plsc.VectorSubcoreMesh(core_axis_name="c", subcore_axis_name="s",
                                  num_cores=NC, num_subcores=NS)

_proj = pl.kernel(
    _proj_body,
    out_type=jax.ShapeDtypeStruct((VOCAB,), jnp.int32),
    mesh=_SC_MESH,
    compiler_params=pltpu.CompilerParams(needs_layout_passes=False),
    scratch_types=[
        pltpu.VMEM((EMB_WORDS,), jnp.float32),
        pltpu.VMEM((EMB_WORDS,), jnp.float32),
        pltpu.VMEM((PROJ_CHUNK,), jnp.int32),
        pltpu.VMEM((2 * EMB,), jnp.float32),
        pltpu.SemaphoreType.DMA,
        pltpu.SemaphoreType.DMA,
    ],
)

_pool = pl.kernel(
    _pool_body,
    out_type=jax.ShapeDtypeStruct((BATCH, OUT), jnp.float32),
    mesh=_SC_MESH,
    compiler_params=pltpu.CompilerParams(needs_layout_passes=False),
    scratch_types=[
        pltpu.VMEM((CHUNK_WORDS + L,), jnp.int32),
        pltpu.VMEM((VOCAB,), jnp.int32),
        pltpu.VMEM((CHUNK_ROWS * L,), jnp.float32),
        pltpu.VMEM((CHUNK_ROWS * L,), jnp.float32),
        pltpu.VMEM((ROWS_PER_W, OUT), jnp.float32),
        pltpu.VMEM((OUT * L,), jnp.float32),
    ],
)


def kernel(text, emb_table, W1, b1, W2, b2):
    wc = pl.pallas_call(
        _wc_body,
        out_shape=jax.ShapeDtypeStruct((OUT, EMB), jnp.float32),
    )(W1, W2)
    p = _proj(emb_table.reshape(-1), wc.reshape(-1))
    bias = W2 @ b1 + b2
    bias_rep = jnp.broadcast_to(bias[:, None], (OUT, L)).reshape(OUT * L)
    return _pool(text.reshape(-1), p, bias_rep)
